# Initial kernel scaffold; baseline (speedup 1.0000x reference)
import jax
import jax.numpy as jnp
from jax.experimental import pallas as pl


def kernel(batch, x, edge_index, pseudo, Wl1, bl1, Wr1, Wl2, bl2, Wr2, Wl3, bl3, Wr3, Wl4, bl4, Wr4, Wl5, bl5, Wr5, Wl6, bl6, Wr6, fc1_W, fc1_b, fc2_W, fc2_b, bn2_g, bn2_b, fc3_W, fc3_b, out_W, out_b):
    raise NotImplementedError("write your pallas kernel here")


# speedup vs baseline: 1.0353x; 1.0353x over previous
import functools

import jax
import jax.numpy as jnp
import numpy as np
from jax import lax
from jax.experimental import pallas as pl
from jax.experimental.pallas import tpu as pltpu

_ROWS = 1000


def _elu(x):
    return jnp.where(x > 0, x, jnp.exp(jnp.minimum(x, 0.0)) - 1.0)


def _sage_body(residual, agg_ref, h_ref, cnt_ref, wl_ref, bl_ref, wr_ref, o_ref):
    inv = 1.0 / jnp.maximum(cnt_ref[...], 1.0)
    mean = agg_ref[...] * inv
    t = (jnp.dot(mean, wl_ref[...], preferred_element_type=jnp.float32)
         + bl_ref[...]
         + jnp.dot(h_ref[...], wr_ref[...], preferred_element_type=jnp.float32))
    a = _elu(t)
    o_ref[...] = a + h_ref[...] if residual else a


def _sage_tc(agg, h, cnt, Wl, bl, Wr, residual):
    n, din = h.shape
    dout = Wl.shape[1]
    grid = n // _ROWS
    return pl.pallas_call(
        functools.partial(_sage_body, residual),
        grid=(grid,),
        in_specs=[
            pl.BlockSpec((_ROWS, din), lambda i: (i, 0)),
            pl.BlockSpec((_ROWS, din), lambda i: (i, 0)),
            pl.BlockSpec((_ROWS, 1), lambda i: (i, 0)),
            pl.BlockSpec((din, dout), lambda i: (0, 0)),
            pl.BlockSpec((1, dout), lambda i: (0, 0)),
            pl.BlockSpec((din, dout), lambda i: (0, 0)),
        ],
        out_specs=pl.BlockSpec((_ROWS, dout), lambda i: (i, 0)),
        out_shape=jax.ShapeDtypeStruct((n, dout), jnp.float32),
    )(agg, h, cnt, Wl, bl, Wr)


def _sage1_body(agg_ref, x_ref, wl_ref, bl_ref, wr_ref, h_ref, cnt_ref):
    agg = agg_ref[...]
    cnt = agg[:, 13:14]
    inv = 1.0 / jnp.maximum(cnt, 1.0)
    mean = agg * inv
    t = (jnp.dot(mean, wl_ref[...], preferred_element_type=jnp.float32)
         + bl_ref[...]
         + jnp.dot(x_ref[...], wr_ref[...], preferred_element_type=jnp.float32))
    h_ref[...] = _elu(t)
    cnt_ref[...] = cnt


def _sage1_tc(agg, xp, Wlp, bl, Wrp):
    n = xp.shape[0]
    dout = Wlp.shape[1]
    grid = n // _ROWS
    return pl.pallas_call(
        _sage1_body,
        grid=(grid,),
        in_specs=[
            pl.BlockSpec((_ROWS, 16), lambda i: (i, 0)),
            pl.BlockSpec((_ROWS, 16), lambda i: (i, 0)),
            pl.BlockSpec((16, dout), lambda i: (0, 0)),
            pl.BlockSpec((1, dout), lambda i: (0, 0)),
            pl.BlockSpec((16, dout), lambda i: (0, 0)),
        ],
        out_specs=[
            pl.BlockSpec((_ROWS, dout), lambda i: (i, 0)),
            pl.BlockSpec((_ROWS, 1), lambda i: (i, 0)),
        ],
        out_shape=[
            jax.ShapeDtypeStruct((n, dout), jnp.float32),
            jax.ShapeDtypeStruct((n, 1), jnp.float32),
        ],
    )(agg, xp, Wlp, bl, Wrp)


def _mlp_body(residual, h_ref, w_ref, b_ref, o_ref):
    t = _elu(jnp.dot(h_ref[...], w_ref[...], preferred_element_type=jnp.float32)
             + b_ref[...])
    o_ref[...] = t + h_ref[...] if residual else t


def _mlp_tc(h, W, b, residual):
    n, din = h.shape
    dout = W.shape[1]
    grid = n // _ROWS
    return pl.pallas_call(
        functools.partial(_mlp_body, residual),
        grid=(grid,),
        in_specs=[
            pl.BlockSpec((_ROWS, din), lambda i: (i, 0)),
            pl.BlockSpec((din, dout), lambda i: (0, 0)),
            pl.BlockSpec((1, dout), lambda i: (0, 0)),
        ],
        out_specs=pl.BlockSpec((_ROWS, dout), lambda i: (i, 0)),
        out_shape=jax.ShapeDtypeStruct((n, dout), jnp.float32),
    )(h, W, b)


def _pool_body(h_ref, b_ref, ps_ref, pc_ref):
    @pl.when(pl.program_id(0) == 0)
    def _init():
        ps_ref[...] = jnp.zeros_like(ps_ref)
        pc_ref[...] = jnp.zeros_like(pc_ref)

    onehot = (b_ref[...] == lax.broadcasted_iota(jnp.int32, (1, 64), 1)
              ).astype(jnp.float32)
    dn = (((0,), (0,)), ((), ()))
    ps_ref[...] += lax.dot_general(onehot, h_ref[...], dn,
                                   preferred_element_type=jnp.float32)
    pc_ref[...] += lax.dot_general(onehot, jnp.ones_like(h_ref[...]), dn,
                                   preferred_element_type=jnp.float32)


def _pool_tc(h, batch2d):
    n, d = h.shape
    grid = n // _ROWS
    return pl.pallas_call(
        _pool_body,
        grid=(grid,),
        in_specs=[
            pl.BlockSpec((_ROWS, d), lambda i: (i, 0)),
            pl.BlockSpec((_ROWS, 1), lambda i: (i, 0)),
        ],
        out_specs=[
            pl.BlockSpec((64, d), lambda i: (0, 0)),
            pl.BlockSpec((64, d), lambda i: (0, 0)),
        ],
        out_shape=[
            jax.ShapeDtypeStruct((64, d), jnp.float32),
            jax.ShapeDtypeStruct((64, d), jnp.float32),
        ],
    )(h, batch2d)


def _tail_body(ps_ref, pc_ref, g_ref, b_ref, w3_ref, b3_ref, wo_ref, bo_ref,
               y_ref):
    pooled = ps_ref[...] / jnp.maximum(pc_ref[...], 1.0)
    y = pooled * np.float32(1.0 / np.sqrt(1.0 + 1e-5)) * g_ref[...] + b_ref[...]
    y = _elu(jnp.dot(y, w3_ref[...], preferred_element_type=jnp.float32)
             + b3_ref[...])
    y = jnp.dot(y, wo_ref[...], preferred_element_type=jnp.float32) + bo_ref[...]
    nrm = jnp.sqrt(jnp.sum(y * y, axis=-1, keepdims=True))
    y_ref[...] = y / jnp.maximum(nrm, 1e-12)


def _tail_tc(ps, pc, bn2_g, bn2_b, fc3_W, fc3_b, out_W, out_b):
    return pl.pallas_call(
        _tail_body,
        out_shape=jax.ShapeDtypeStruct((64, 3), jnp.float32),
    )(ps, pc, bn2_g, bn2_b, fc3_W, fc3_b, out_W, out_b)


def _agg(h, src, dst, n):
    return jax.ops.segment_sum(jnp.take(h, src, axis=0), dst, num_segments=n)


def kernel(batch, x, edge_index, pseudo, Wl1, bl1, Wr1, Wl2, bl2, Wr2, Wl3,
           bl3, Wr3, Wl4, bl4, Wr4, Wl5, bl5, Wr5, Wl6, bl6, Wr6, fc1_W,
           fc1_b, fc2_W, fc2_b, bn2_g, bn2_b, fc3_W, fc3_b, out_W, out_b):
    n = x.shape[0]
    src = edge_index[0]
    dst = edge_index[1]

    xp = jnp.concatenate(
        [x, jnp.ones((n, 1), jnp.float32), jnp.zeros((n, 2), jnp.float32)],
        axis=1)
    Wl1p = jnp.concatenate([Wl1, jnp.zeros((3, Wl1.shape[1]), jnp.float32)], 0)
    Wr1p = jnp.concatenate([Wr1, jnp.zeros((3, Wr1.shape[1]), jnp.float32)], 0)

    agg1 = _agg(xp, src, dst, n)
    h, cnt = _sage1_tc(agg1, xp, Wl1p, bl1.reshape(1, -1), Wr1p)

    for Wl, bl, Wr, res in (
        (Wl2, bl2, Wr2, False),
        (Wl3, bl3, Wr3, True),
        (Wl4, bl4, Wr4, True),
        (Wl5, bl5, Wr5, True),
        (Wl6, bl6, Wr6, True),
    ):
        agg = _agg(h, src, dst, n)
        h = _sage_tc(agg, h, cnt, Wl, bl.reshape(1, -1), Wr, res)

    h = _mlp_tc(h, fc1_W, fc1_b.reshape(1, -1), True)
    h = _mlp_tc(h, fc2_W, fc2_b.reshape(1, -1), False)

    ps, pc = _pool_tc(h, batch.reshape(-1, 1))
    return _tail_tc(ps, pc, bn2_g.reshape(1, -1), bn2_b.reshape(1, -1),
                    fc3_W, fc3_b.reshape(1, -1), out_W, out_b.reshape(1, -1))


# speedup vs baseline: 1.9144x; 1.8490x over previous
import functools

import jax
import jax.numpy as jnp
import numpy as np
from jax import lax
from jax.experimental import pallas as pl
from jax.experimental.pallas import tpu as pltpu
from jax.experimental.pallas import tpu_sc as plsc

_ROWS = 1000

_NC = 2
_NS = 16
_K = 128
_NPADROWS = 48
_NDUMMY = 16


def _elu(x):
    return jnp.where(x > 0, x, jnp.exp(jnp.minimum(x, 0.0)) - 1.0)


def _sage_body(residual, agg_ref, h_ref, cnt_ref, wl_ref, bl_ref, wr_ref, o_ref):
    inv = 1.0 / jnp.maximum(cnt_ref[...], 1.0)
    mean = agg_ref[...] * inv
    t = (jnp.dot(mean, wl_ref[...], preferred_element_type=jnp.float32)
         + bl_ref[...]
         + jnp.dot(h_ref[...], wr_ref[...], preferred_element_type=jnp.float32))
    a = _elu(t)
    o_ref[...] = a + h_ref[...] if residual else a


def _sage_tc(agg, h, cnt, Wl, bl, Wr, residual):
    n, din = h.shape
    dout = Wl.shape[1]
    grid = n // _ROWS
    return pl.pallas_call(
        functools.partial(_sage_body, residual),
        grid=(grid,),
        in_specs=[
            pl.BlockSpec((_ROWS, din), lambda i: (i, 0)),
            pl.BlockSpec((_ROWS, din), lambda i: (i, 0)),
            pl.BlockSpec((_ROWS, 1), lambda i: (i, 0)),
            pl.BlockSpec((din, dout), lambda i: (0, 0)),
            pl.BlockSpec((1, dout), lambda i: (0, 0)),
            pl.BlockSpec((din, dout), lambda i: (0, 0)),
        ],
        out_specs=pl.BlockSpec((_ROWS, dout), lambda i: (i, 0)),
        out_shape=jax.ShapeDtypeStruct((n, dout), jnp.float32),
    )(agg, h, cnt, Wl, bl, Wr)


def _sage1_body(agg_ref, x_ref, wl_ref, bl_ref, wr_ref, h_ref, cnt_ref):
    agg = agg_ref[...]
    cnt = agg[:, 13:14]
    inv = 1.0 / jnp.maximum(cnt, 1.0)
    mean = agg * inv
    t = (jnp.dot(mean, wl_ref[...], preferred_element_type=jnp.float32)
         + bl_ref[...]
         + jnp.dot(x_ref[...], wr_ref[...], preferred_element_type=jnp.float32))
    h_ref[...] = _elu(t)
    cnt_ref[...] = cnt


def _sage1_tc(agg, xp, Wlp, bl, Wrp):
    n = xp.shape[0]
    dout = Wlp.shape[1]
    grid = n // _ROWS
    return pl.pallas_call(
        _sage1_body,
        grid=(grid,),
        in_specs=[
            pl.BlockSpec((_ROWS, 16), lambda i: (i, 0)),
            pl.BlockSpec((_ROWS, 16), lambda i: (i, 0)),
            pl.BlockSpec((16, dout), lambda i: (0, 0)),
            pl.BlockSpec((1, dout), lambda i: (0, 0)),
            pl.BlockSpec((16, dout), lambda i: (0, 0)),
        ],
        out_specs=[
            pl.BlockSpec((_ROWS, dout), lambda i: (i, 0)),
            pl.BlockSpec((_ROWS, 1), lambda i: (i, 0)),
        ],
        out_shape=[
            jax.ShapeDtypeStruct((n, dout), jnp.float32),
            jax.ShapeDtypeStruct((n, 1), jnp.float32),
        ],
    )(agg, xp, Wlp, bl, Wrp)


def _mlp_body(residual, h_ref, w_ref, b_ref, o_ref):
    t = _elu(jnp.dot(h_ref[...], w_ref[...], preferred_element_type=jnp.float32)
             + b_ref[...])
    o_ref[...] = t + h_ref[...] if residual else t


def _mlp_tc(h, W, b, residual):
    n, din = h.shape
    dout = W.shape[1]
    grid = n // _ROWS
    return pl.pallas_call(
        functools.partial(_mlp_body, residual),
        grid=(grid,),
        in_specs=[
            pl.BlockSpec((_ROWS, din), lambda i: (i, 0)),
            pl.BlockSpec((din, dout), lambda i: (0, 0)),
            pl.BlockSpec((1, dout), lambda i: (0, 0)),
        ],
        out_specs=pl.BlockSpec((_ROWS, dout), lambda i: (i, 0)),
        out_shape=jax.ShapeDtypeStruct((n, dout), jnp.float32),
    )(h, W, b)


def _pool_body(h_ref, b_ref, ps_ref, pc_ref):
    @pl.when(pl.program_id(0) == 0)
    def _init():
        ps_ref[...] = jnp.zeros_like(ps_ref)
        pc_ref[...] = jnp.zeros_like(pc_ref)

    onehot = (b_ref[...] == lax.broadcasted_iota(jnp.int32, (1, 64), 1)
              ).astype(jnp.float32)
    dn = (((0,), (0,)), ((), ()))
    ps_ref[...] += lax.dot_general(onehot, h_ref[...], dn,
                                   preferred_element_type=jnp.float32)
    pc_ref[...] += lax.dot_general(onehot, jnp.ones_like(h_ref[...]), dn,
                                   preferred_element_type=jnp.float32)


def _pool_tc(h, batch2d):
    n, d = h.shape
    grid = n // _ROWS
    return pl.pallas_call(
        _pool_body,
        grid=(grid,),
        in_specs=[
            pl.BlockSpec((_ROWS, d), lambda i: (i, 0)),
            pl.BlockSpec((_ROWS, 1), lambda i: (i, 0)),
        ],
        out_specs=[
            pl.BlockSpec((64, d), lambda i: (0, 0)),
            pl.BlockSpec((64, d), lambda i: (0, 0)),
        ],
        out_shape=[
            jax.ShapeDtypeStruct((64, d), jnp.float32),
            jax.ShapeDtypeStruct((64, d), jnp.float32),
        ],
    )(h, batch2d)


def _tail_body(ps_ref, pc_ref, g_ref, b_ref, w3_ref, b3_ref, wo_ref, bo_ref,
               y_ref):
    pooled = ps_ref[...] / jnp.maximum(pc_ref[...], 1.0)
    y = pooled * np.float32(1.0 / np.sqrt(1.0 + 1e-5)) * g_ref[...] + b_ref[...]
    y = _elu(jnp.dot(y, w3_ref[...], preferred_element_type=jnp.float32)
             + b3_ref[...])
    y = jnp.dot(y, wo_ref[...], preferred_element_type=jnp.float32) + bo_ref[...]
    nrm = jnp.sqrt(jnp.sum(y * y, axis=-1, keepdims=True))
    y_ref[...] = y / jnp.maximum(nrm, 1e-12)


def _tail_tc(ps, pc, bn2_g, bn2_b, fc3_W, fc3_b, out_W, out_b):
    return pl.pallas_call(
        _tail_body,
        out_shape=jax.ShapeDtypeStruct((64, 3), jnp.float32),
    )(ps, pc, bn2_g, bn2_b, fc3_W, fc3_b, out_W, out_b)


def _sc_agg_kernel(h3, src_pad, dst_pad, zeros_pad, dc):
    nf, n, _ = h3.shape
    npad = n + _NPADROWS
    ep = src_pad.shape[0]
    ec = ep // _NC
    et = ec // _NS
    nblk = et // _K
    nz = npad // _NS
    no = npad // _NS
    nlast = n - (_NS - 1) * no

    mesh = plsc.VectorSubcoreMesh(core_axis_name="c", subcore_axis_name="s",
                                  num_cores=_NC, num_subcores=_NS)

    def body(h_hbm, src_hbm, dst_hbm, z_hbm, out_hbm, acc, sidx, didx, stage,
             gsem, ssem):
        cid = lax.axis_index("c")
        sid = lax.axis_index("s")
        ebase = cid * ec + sid * et
        for f in range(nf):
            tbl = h_hbm.at[f]
            pltpu.sync_copy(z_hbm.at[pl.ds(sid * nz, nz)],
                            acc.at[pl.ds(sid * nz, nz)])
            plsc.subcore_barrier()

            def blk(b, carry):
                e0 = ebase + b * _K
                pltpu.sync_copy(src_hbm.at[pl.ds(e0, _K)], sidx)
                pltpu.sync_copy(dst_hbm.at[pl.ds(e0, _K)], didx)
                pltpu.async_copy(tbl.at[sidx], stage, gsem).wait()
                pltpu.async_copy(stage, acc.at[didx], ssem, add=True).wait()
                return carry

            lax.fori_loop(0, nblk, blk, 0)
            plsc.subcore_barrier()

            @pl.when(sid < _NS - 1)
            def _copy_full():
                pltpu.sync_copy(acc.at[pl.ds(sid * no, no)],
                                out_hbm.at[cid, f, pl.ds(sid * no, no)])

            @pl.when(sid == _NS - 1)
            def _copy_last():
                pltpu.sync_copy(
                    acc.at[pl.ds((_NS - 1) * no, nlast)],
                    out_hbm.at[cid, f, pl.ds((_NS - 1) * no, nlast)])

            plsc.subcore_barrier()

    run = pl.kernel(
        body,
        out_type=jax.ShapeDtypeStruct((_NC, nf, n, dc), jnp.float32),
        mesh=mesh,
        compiler_params=pltpu.CompilerParams(use_tc_tiling_on_sc=False),
        scratch_types=[
            pltpu.VMEM_SHARED((npad, dc), jnp.float32),
            pltpu.VMEM((_K,), jnp.int32),
            pltpu.VMEM((_K,), jnp.int32),
            pltpu.VMEM((_K, dc), jnp.float32),
            pltpu.SemaphoreType.DMA,
            pltpu.SemaphoreType.DMA,
        ],
    )
    return run(h3, src_pad, dst_pad, zeros_pad)


def _sc_agg(h, src_pad, dst_pad, zeros_pad, dc):
    n, d = h.shape
    nf = d // dc
    h3 = h.reshape(n, nf, dc).transpose(1, 0, 2)
    out = _sc_agg_kernel(h3, src_pad, dst_pad, zeros_pad, dc)
    return (out[0] + out[1]).transpose(1, 0, 2).reshape(n, d)


def kernel(batch, x, edge_index, pseudo, Wl1, bl1, Wr1, Wl2, bl2, Wr2, Wl3,
           bl3, Wr3, Wl4, bl4, Wr4, Wl5, bl5, Wr5, Wl6, bl6, Wr6, fc1_W,
           fc1_b, fc2_W, fc2_b, bn2_g, bn2_b, fc3_W, fc3_b, out_W, out_b):
    n = x.shape[0]
    e = edge_index.shape[1]
    src = edge_index[0]
    dst = edge_index[1]

    grain = _NC * _NS * 2 * _K
    ep = ((e + grain - 1) // grain) * grain
    padn = ep - e
    pada = jnp.arange(padn, dtype=jnp.int32)
    srcp = jnp.concatenate([src, (pada * 997) % n])
    dstp = jnp.concatenate([dst, n + (pada % _NDUMMY)])
    z16 = jnp.zeros((n + _NPADROWS, 16), jnp.float32)
    z32 = jnp.zeros((n + _NPADROWS, 32), jnp.float32)

    xp = jnp.concatenate(
        [x, jnp.ones((n, 1), jnp.float32), jnp.zeros((n, 2), jnp.float32)],
        axis=1)
    Wl1p = jnp.concatenate([Wl1, jnp.zeros((3, Wl1.shape[1]), jnp.float32)], 0)
    Wr1p = jnp.concatenate([Wr1, jnp.zeros((3, Wr1.shape[1]), jnp.float32)], 0)

    agg1 = _sc_agg(xp, srcp, dstp, z16, 16)
    h, cnt = _sage1_tc(agg1, xp, Wl1p, bl1.reshape(1, -1), Wr1p)

    for Wl, bl, Wr, res in (
        (Wl2, bl2, Wr2, False),
        (Wl3, bl3, Wr3, True),
        (Wl4, bl4, Wr4, True),
        (Wl5, bl5, Wr5, True),
        (Wl6, bl6, Wr6, True),
    ):
        agg = _sc_agg(h, srcp, dstp, z32, 32)
        h = _sage_tc(agg, h, cnt, Wl, bl.reshape(1, -1), Wr, res)

    h = _mlp_tc(h, fc1_W, fc1_b.reshape(1, -1), True)
    h = _mlp_tc(h, fc2_W, fc2_b.reshape(1, -1), False)

    ps, pc = _pool_tc(h, batch.reshape(-1, 1))
    return _tail_tc(ps, pc, bn2_g.reshape(1, -1), bn2_b.reshape(1, -1),
                    fc3_W, fc3_b.reshape(1, -1), out_W, out_b.reshape(1, -1))


# speedup vs baseline: 3.8811x; 2.0273x over previous
import functools

import jax
import jax.numpy as jnp
import numpy as np
from jax import lax
from jax.experimental import pallas as pl
from jax.experimental.pallas import tpu as pltpu
from jax.experimental.pallas import tpu_sc as plsc

_ROWS = 1000

_NC = 2
_NS = 16
_K = 128
_NPADROWS = 48
_NDUMMY = 16


def _elu(x):
    return jnp.where(x > 0, x, jnp.exp(jnp.minimum(x, 0.0)) - 1.0)


def _sage_body(residual, agg_ref, h_ref, cnt_ref, wl_ref, bl_ref, wr_ref, o_ref):
    inv = 1.0 / jnp.maximum(cnt_ref[...], 1.0)
    mean = agg_ref[...] * inv
    t = (jnp.dot(mean, wl_ref[...], preferred_element_type=jnp.float32)
         + bl_ref[...]
         + jnp.dot(h_ref[...], wr_ref[...], preferred_element_type=jnp.float32))
    a = _elu(t)
    o_ref[...] = a + h_ref[...] if residual else a


def _sage_tc(agg, h, cnt, Wl, bl, Wr, residual):
    n, din = h.shape
    dout = Wl.shape[1]
    grid = n // _ROWS
    return pl.pallas_call(
        functools.partial(_sage_body, residual),
        grid=(grid,),
        in_specs=[
            pl.BlockSpec((_ROWS, din), lambda i: (i, 0)),
            pl.BlockSpec((_ROWS, din), lambda i: (i, 0)),
            pl.BlockSpec((_ROWS, 1), lambda i: (i, 0)),
            pl.BlockSpec((din, dout), lambda i: (0, 0)),
            pl.BlockSpec((1, dout), lambda i: (0, 0)),
            pl.BlockSpec((din, dout), lambda i: (0, 0)),
        ],
        out_specs=pl.BlockSpec((_ROWS, dout), lambda i: (i, 0)),
        out_shape=jax.ShapeDtypeStruct((n, dout), jnp.float32),
    )(agg, h, cnt, Wl, bl, Wr)


def _sage1_body(agg_ref, x_ref, wl_ref, bl_ref, wr_ref, h_ref, cnt_ref):
    agg = agg_ref[...]
    cnt = agg[:, 13:14]
    inv = 1.0 / jnp.maximum(cnt, 1.0)
    mean = agg * inv
    t = (jnp.dot(mean, wl_ref[...], preferred_element_type=jnp.float32)
         + bl_ref[...]
         + jnp.dot(x_ref[...], wr_ref[...], preferred_element_type=jnp.float32))
    h_ref[...] = _elu(t)
    cnt_ref[...] = cnt


def _sage1_tc(agg, xp, Wlp, bl, Wrp):
    n = xp.shape[0]
    dout = Wlp.shape[1]
    grid = n // _ROWS
    return pl.pallas_call(
        _sage1_body,
        grid=(grid,),
        in_specs=[
            pl.BlockSpec((_ROWS, 16), lambda i: (i, 0)),
            pl.BlockSpec((_ROWS, 16), lambda i: (i, 0)),
            pl.BlockSpec((16, dout), lambda i: (0, 0)),
            pl.BlockSpec((1, dout), lambda i: (0, 0)),
            pl.BlockSpec((16, dout), lambda i: (0, 0)),
        ],
        out_specs=[
            pl.BlockSpec((_ROWS, dout), lambda i: (i, 0)),
            pl.BlockSpec((_ROWS, 1), lambda i: (i, 0)),
        ],
        out_shape=[
            jax.ShapeDtypeStruct((n, dout), jnp.float32),
            jax.ShapeDtypeStruct((n, 1), jnp.float32),
        ],
    )(agg, xp, Wlp, bl, Wrp)


def _mlp_body(residual, h_ref, w_ref, b_ref, o_ref):
    t = _elu(jnp.dot(h_ref[...], w_ref[...], preferred_element_type=jnp.float32)
             + b_ref[...])
    o_ref[...] = t + h_ref[...] if residual else t


def _mlp_tc(h, W, b, residual):
    n, din = h.shape
    dout = W.shape[1]
    grid = n // _ROWS
    return pl.pallas_call(
        functools.partial(_mlp_body, residual),
        grid=(grid,),
        in_specs=[
            pl.BlockSpec((_ROWS, din), lambda i: (i, 0)),
            pl.BlockSpec((din, dout), lambda i: (0, 0)),
            pl.BlockSpec((1, dout), lambda i: (0, 0)),
        ],
        out_specs=pl.BlockSpec((_ROWS, dout), lambda i: (i, 0)),
        out_shape=jax.ShapeDtypeStruct((n, dout), jnp.float32),
    )(h, W, b)


def _pool_body(h_ref, b_ref, ps_ref, pc_ref):
    @pl.when(pl.program_id(0) == 0)
    def _init():
        ps_ref[...] = jnp.zeros_like(ps_ref)
        pc_ref[...] = jnp.zeros_like(pc_ref)

    onehot = (b_ref[...] == lax.broadcasted_iota(jnp.int32, (1, 64), 1)
              ).astype(jnp.float32)
    dn = (((0,), (0,)), ((), ()))
    ps_ref[...] += lax.dot_general(onehot, h_ref[...], dn,
                                   preferred_element_type=jnp.float32)
    pc_ref[...] += lax.dot_general(onehot, jnp.ones_like(h_ref[...]), dn,
                                   preferred_element_type=jnp.float32)


def _pool_tc(h, batch2d):
    n, d = h.shape
    grid = n // _ROWS
    return pl.pallas_call(
        _pool_body,
        grid=(grid,),
        in_specs=[
            pl.BlockSpec((_ROWS, d), lambda i: (i, 0)),
            pl.BlockSpec((_ROWS, 1), lambda i: (i, 0)),
        ],
        out_specs=[
            pl.BlockSpec((64, d), lambda i: (0, 0)),
            pl.BlockSpec((64, d), lambda i: (0, 0)),
        ],
        out_shape=[
            jax.ShapeDtypeStruct((64, d), jnp.float32),
            jax.ShapeDtypeStruct((64, d), jnp.float32),
        ],
    )(h, batch2d)


def _tail_body(ps_ref, pc_ref, g_ref, b_ref, w3_ref, b3_ref, wo_ref, bo_ref,
               y_ref):
    pooled = ps_ref[...] / jnp.maximum(pc_ref[...], 1.0)
    y = pooled * np.float32(1.0 / np.sqrt(1.0 + 1e-5)) * g_ref[...] + b_ref[...]
    y = _elu(jnp.dot(y, w3_ref[...], preferred_element_type=jnp.float32)
             + b3_ref[...])
    y = jnp.dot(y, wo_ref[...], preferred_element_type=jnp.float32) + bo_ref[...]
    nrm = jnp.sqrt(jnp.sum(y * y, axis=-1, keepdims=True))
    y_ref[...] = y / jnp.maximum(nrm, 1e-12)


def _tail_tc(ps, pc, bn2_g, bn2_b, fc3_W, fc3_b, out_W, out_b):
    return pl.pallas_call(
        _tail_body,
        out_shape=jax.ShapeDtypeStruct((64, 3), jnp.float32),
    )(ps, pc, bn2_g, bn2_b, fc3_W, fc3_b, out_W, out_b)


def _sc_agg_kernel(h3, idx3, zeros_pad, dc):
    nf, n, _ = h3.shape
    npad = n + _NPADROWS
    nbt = idx3.shape[0]
    nblk = nbt // (_NC * _NS)
    half = nblk // 2
    nz = npad // _NS
    no = npad // _NS
    nlast = n - (_NS - 1) * no

    mesh = plsc.VectorSubcoreMesh(core_axis_name="c", subcore_axis_name="s",
                                  num_cores=_NC, num_subcores=_NS)

    def body(h_hbm, idx_hbm, z_hbm, out_hbm, acc, eb0, eb1, stage0, stage1,
             gsem0, gsem1, ssem0, ssem1):
        cid = lax.axis_index("c")
        sid = lax.axis_index("s")
        bbase = (cid * _NS + sid) * nblk
        for f in range(nf):
            tbl = h_hbm.at[f]
            pltpu.sync_copy(z_hbm.at[pl.ds(sid * nz, nz)],
                            acc.at[pl.ds(sid * nz, nz)])
            plsc.subcore_barrier()

            def drain(sem):
                pltpu.make_async_copy(tbl.at[pl.ds(0, _K)], stage0, sem).wait()

            pltpu.sync_copy(idx_hbm.at[bbase], eb0)
            pltpu.async_copy(tbl.at[eb0.at[0]], stage0, gsem0)

            def blk(i, carry):
                @pl.when(i > 0)
                def _():
                    drain(ssem1)

                pltpu.sync_copy(idx_hbm.at[bbase + 2 * i + 1], eb1)
                pltpu.async_copy(tbl.at[eb1.at[0]], stage1, gsem1)
                drain(gsem0)
                pltpu.async_copy(stage0, acc.at[eb0.at[1]], ssem0, add=True)

                @pl.when(i < half - 1)
                def _():
                    drain(ssem0)
                    pltpu.sync_copy(idx_hbm.at[bbase + 2 * i + 2], eb0)
                    pltpu.async_copy(tbl.at[eb0.at[0]], stage0, gsem0)

                drain(gsem1)
                pltpu.async_copy(stage1, acc.at[eb1.at[1]], ssem1, add=True)
                return carry

            lax.fori_loop(0, half, blk, 0)
            drain(ssem0)
            drain(ssem1)
            plsc.subcore_barrier()

            @pl.when(sid < _NS - 1)
            def _copy_full():
                pltpu.sync_copy(acc.at[pl.ds(sid * no, no)],
                                out_hbm.at[cid, f, pl.ds(sid * no, no)])

            @pl.when(sid == _NS - 1)
            def _copy_last():
                pltpu.sync_copy(
                    acc.at[pl.ds((_NS - 1) * no, nlast)],
                    out_hbm.at[cid, f, pl.ds((_NS - 1) * no, nlast)])

            plsc.subcore_barrier()

    run = pl.kernel(
        body,
        out_type=jax.ShapeDtypeStruct((_NC, nf, n, dc), jnp.float32),
        mesh=mesh,
        compiler_params=pltpu.CompilerParams(use_tc_tiling_on_sc=False),
        scratch_types=[
            pltpu.VMEM_SHARED((npad, dc), jnp.float32),
            pltpu.VMEM((2, _K), jnp.int32),
            pltpu.VMEM((2, _K), jnp.int32),
            pltpu.VMEM((_K, dc), jnp.float32),
            pltpu.VMEM((_K, dc), jnp.float32),
            pltpu.SemaphoreType.DMA,
            pltpu.SemaphoreType.DMA,
            pltpu.SemaphoreType.DMA,
            pltpu.SemaphoreType.DMA,
        ],
    )
    return run(h3, idx3, zeros_pad)


def _sc_agg(h, idx3, zeros_pad, dc):
    n, d = h.shape
    nf = d // dc
    h3 = h.reshape(n, nf, dc).transpose(1, 0, 2)
    out = _sc_agg_kernel(h3, idx3, zeros_pad, dc)
    return (out[0] + out[1]).transpose(1, 0, 2).reshape(n, d)


def kernel(batch, x, edge_index, pseudo, Wl1, bl1, Wr1, Wl2, bl2, Wr2, Wl3,
           bl3, Wr3, Wl4, bl4, Wr4, Wl5, bl5, Wr5, Wl6, bl6, Wr6, fc1_W,
           fc1_b, fc2_W, fc2_b, bn2_g, bn2_b, fc3_W, fc3_b, out_W, out_b):
    n = x.shape[0]
    e = edge_index.shape[1]
    src = edge_index[0]
    dst = edge_index[1]

    grain = _NC * _NS * 2 * _K
    ep = ((e + grain - 1) // grain) * grain
    padn = ep - e
    pada = jnp.arange(padn, dtype=jnp.int32)
    srcp = jnp.concatenate([src, (pada * 997) % n])
    dstp = jnp.concatenate([dst, n + (pada % _NDUMMY)])
    idx3 = jnp.stack([srcp.reshape(-1, _K), dstp.reshape(-1, _K)], axis=1)
    z16 = jnp.zeros((n + _NPADROWS, 16), jnp.float32)
    z32 = jnp.zeros((n + _NPADROWS, 32), jnp.float32)

    xp = jnp.concatenate(
        [x, jnp.ones((n, 1), jnp.float32), jnp.zeros((n, 2), jnp.float32)],
        axis=1)
    Wl1p = jnp.concatenate([Wl1, jnp.zeros((3, Wl1.shape[1]), jnp.float32)], 0)
    Wr1p = jnp.concatenate([Wr1, jnp.zeros((3, Wr1.shape[1]), jnp.float32)], 0)

    agg1 = _sc_agg(xp, idx3, z16, 16)
    h, cnt = _sage1_tc(agg1, xp, Wl1p, bl1.reshape(1, -1), Wr1p)

    for Wl, bl, Wr, res in (
        (Wl2, bl2, Wr2, False),
        (Wl3, bl3, Wr3, True),
        (Wl4, bl4, Wr4, True),
        (Wl5, bl5, Wr5, True),
        (Wl6, bl6, Wr6, True),
    ):
        agg = _sc_agg(h, idx3, z32, 32)
        h = _sage_tc(agg, h, cnt, Wl, bl.reshape(1, -1), Wr, res)

    h = _mlp_tc(h, fc1_W, fc1_b.reshape(1, -1), True)
    h = _mlp_tc(h, fc2_W, fc2_b.reshape(1, -1), False)

    ps, pc = _pool_tc(h, batch.reshape(-1, 1))
    return _tail_tc(ps, pc, bn2_g.reshape(1, -1), bn2_b.reshape(1, -1),
                    fc3_W, fc3_b.reshape(1, -1), out_W, out_b.reshape(1, -1))


# speedup vs baseline: 5.1855x; 1.3361x over previous
import functools

import jax
import jax.numpy as jnp
import numpy as np
from jax import lax
from jax.experimental import pallas as pl
from jax.experimental.pallas import tpu as pltpu
from jax.experimental.pallas import tpu_sc as plsc

_ROWS = 1000

_NC = 2
_NS = 16
_K = 128
_U = 2
_NPADROWS = 48
_NDUMMY = 16


def _elu(x):
    return jnp.where(x > 0, x, jnp.exp(jnp.minimum(x, 0.0)) - 1.0)


def _sage_body(residual, agg_ref, h_ref, cnt_ref, wl_ref, bl_ref, wr_ref, o_ref):
    inv = 1.0 / jnp.maximum(cnt_ref[...], 1.0)
    mean = agg_ref[...] * inv
    t = (jnp.dot(mean, wl_ref[...], preferred_element_type=jnp.float32)
         + bl_ref[...]
         + jnp.dot(h_ref[...], wr_ref[...], preferred_element_type=jnp.float32))
    a = _elu(t)
    o_ref[...] = a + h_ref[...] if residual else a


def _sage_tc(agg, h, cnt, Wl, bl, Wr, residual):
    n, din = h.shape
    dout = Wl.shape[1]
    grid = n // _ROWS
    return pl.pallas_call(
        functools.partial(_sage_body, residual),
        grid=(grid,),
        in_specs=[
            pl.BlockSpec((_ROWS, din), lambda i: (i, 0)),
            pl.BlockSpec((_ROWS, din), lambda i: (i, 0)),
            pl.BlockSpec((_ROWS, 1), lambda i: (i, 0)),
            pl.BlockSpec((din, dout), lambda i: (0, 0)),
            pl.BlockSpec((1, dout), lambda i: (0, 0)),
            pl.BlockSpec((din, dout), lambda i: (0, 0)),
        ],
        out_specs=pl.BlockSpec((_ROWS, dout), lambda i: (i, 0)),
        out_shape=jax.ShapeDtypeStruct((n, dout), jnp.float32),
    )(agg, h, cnt, Wl, bl, Wr)


def _sage1_body(agg_ref, x_ref, wl_ref, bl_ref, wr_ref, h_ref, cnt_ref):
    agg = agg_ref[...]
    cnt = agg[:, 13:14]
    inv = 1.0 / jnp.maximum(cnt, 1.0)
    mean = agg * inv
    t = (jnp.dot(mean, wl_ref[...], preferred_element_type=jnp.float32)
         + bl_ref[...]
         + jnp.dot(x_ref[...], wr_ref[...], preferred_element_type=jnp.float32))
    h_ref[...] = _elu(t)
    cnt_ref[...] = cnt


def _sage1_tc(agg, xp, Wlp, bl, Wrp):
    n = xp.shape[0]
    dout = Wlp.shape[1]
    grid = n // _ROWS
    return pl.pallas_call(
        _sage1_body,
        grid=(grid,),
        in_specs=[
            pl.BlockSpec((_ROWS, 16), lambda i: (i, 0)),
            pl.BlockSpec((_ROWS, 16), lambda i: (i, 0)),
            pl.BlockSpec((16, dout), lambda i: (0, 0)),
            pl.BlockSpec((1, dout), lambda i: (0, 0)),
            pl.BlockSpec((16, dout), lambda i: (0, 0)),
        ],
        out_specs=[
            pl.BlockSpec((_ROWS, dout), lambda i: (i, 0)),
            pl.BlockSpec((_ROWS, 1), lambda i: (i, 0)),
        ],
        out_shape=[
            jax.ShapeDtypeStruct((n, dout), jnp.float32),
            jax.ShapeDtypeStruct((n, 1), jnp.float32),
        ],
    )(agg, xp, Wlp, bl, Wrp)


def _mlp_body(residual, h_ref, w_ref, b_ref, o_ref):
    t = _elu(jnp.dot(h_ref[...], w_ref[...], preferred_element_type=jnp.float32)
             + b_ref[...])
    o_ref[...] = t + h_ref[...] if residual else t


def _mlp_tc(h, W, b, residual):
    n, din = h.shape
    dout = W.shape[1]
    grid = n // _ROWS
    return pl.pallas_call(
        functools.partial(_mlp_body, residual),
        grid=(grid,),
        in_specs=[
            pl.BlockSpec((_ROWS, din), lambda i: (i, 0)),
            pl.BlockSpec((din, dout), lambda i: (0, 0)),
            pl.BlockSpec((1, dout), lambda i: (0, 0)),
        ],
        out_specs=pl.BlockSpec((_ROWS, dout), lambda i: (i, 0)),
        out_shape=jax.ShapeDtypeStruct((n, dout), jnp.float32),
    )(h, W, b)


def _pool_body(h_ref, b_ref, ps_ref, pc_ref):
    @pl.when(pl.program_id(0) == 0)
    def _init():
        ps_ref[...] = jnp.zeros_like(ps_ref)
        pc_ref[...] = jnp.zeros_like(pc_ref)

    onehot = (b_ref[...] == lax.broadcasted_iota(jnp.int32, (1, 64), 1)
              ).astype(jnp.float32)
    dn = (((0,), (0,)), ((), ()))
    ps_ref[...] += lax.dot_general(onehot, h_ref[...], dn,
                                   preferred_element_type=jnp.float32)
    pc_ref[...] += lax.dot_general(onehot, jnp.ones_like(h_ref[...]), dn,
                                   preferred_element_type=jnp.float32)


def _pool_tc(h, batch2d):
    n, d = h.shape
    grid = n // _ROWS
    return pl.pallas_call(
        _pool_body,
        grid=(grid,),
        in_specs=[
            pl.BlockSpec((_ROWS, d), lambda i: (i, 0)),
            pl.BlockSpec((_ROWS, 1), lambda i: (i, 0)),
        ],
        out_specs=[
            pl.BlockSpec((64, d), lambda i: (0, 0)),
            pl.BlockSpec((64, d), lambda i: (0, 0)),
        ],
        out_shape=[
            jax.ShapeDtypeStruct((64, d), jnp.float32),
            jax.ShapeDtypeStruct((64, d), jnp.float32),
        ],
    )(h, batch2d)


def _tail_body(ps_ref, pc_ref, g_ref, b_ref, w3_ref, b3_ref, wo_ref, bo_ref,
               y_ref):
    pooled = ps_ref[...] / jnp.maximum(pc_ref[...], 1.0)
    y = pooled * np.float32(1.0 / np.sqrt(1.0 + 1e-5)) * g_ref[...] + b_ref[...]
    y = _elu(jnp.dot(y, w3_ref[...], preferred_element_type=jnp.float32)
             + b3_ref[...])
    y = jnp.dot(y, wo_ref[...], preferred_element_type=jnp.float32) + bo_ref[...]
    nrm = jnp.sqrt(jnp.sum(y * y, axis=-1, keepdims=True))
    y_ref[...] = y / jnp.maximum(nrm, 1e-12)


def _tail_tc(ps, pc, bn2_g, bn2_b, fc3_W, fc3_b, out_W, out_b):
    return pl.pallas_call(
        _tail_body,
        out_shape=jax.ShapeDtypeStruct((64, 3), jnp.float32),
    )(ps, pc, bn2_g, bn2_b, fc3_W, fc3_b, out_W, out_b)


def _sc_agg_kernel(h3, idx3, zeros_pad, dc):
    nf, n, _ = h3.shape
    npad = n + _NPADROWS
    nbt = idx3.shape[0]
    nblk = nbt // (_NC * _NS)
    half = nblk // (2 * _U)
    nz = npad // _NS
    no = npad // _NS
    nlast = n - (_NS - 1) * no

    mesh = plsc.VectorSubcoreMesh(core_axis_name="c", subcore_axis_name="s",
                                  num_cores=_NC, num_subcores=_NS)

    def body(h_hbm, idx_hbm, z_hbm, out_hbm, acc, eb0, eb1, stage0, stage1,
             gsem0, gsem1, ssem0, ssem1):
        cid = lax.axis_index("c")
        sid = lax.axis_index("s")
        bbase = (cid * _NS + sid) * nblk
        for f in range(nf):
            tbl = h_hbm.at[f]
            pltpu.sync_copy(z_hbm.at[pl.ds(sid * nz, nz)],
                            acc.at[pl.ds(sid * nz, nz)])
            plsc.subcore_barrier()

            def drain_unit(sem, stage):
                pltpu.make_async_copy(tbl.at[pl.ds(0, _U * _K)], stage,
                                      sem).wait()

            def fire_gathers(eb, stage, sem):
                for j in range(_U):
                    pltpu.async_copy(tbl.at[eb.at[j, 0]],
                                     stage.at[pl.ds(j * _K, _K)], sem)

            def fire_scatters(eb, stage, sem):
                for j in range(_U):
                    pltpu.async_copy(stage.at[pl.ds(j * _K, _K)],
                                     acc.at[eb.at[j, 1]], sem, add=True)

            pltpu.sync_copy(idx_hbm.at[pl.ds(bbase, _U)], eb0)
            fire_gathers(eb0, stage0, gsem0)

            def blk(i, carry):
                u0 = bbase + 2 * i * _U

                @pl.when(i > 0)
                def _():
                    drain_unit(ssem1, stage1)

                pltpu.sync_copy(idx_hbm.at[pl.ds(u0 + _U, _U)], eb1)
                fire_gathers(eb1, stage1, gsem1)
                drain_unit(gsem0, stage0)
                fire_scatters(eb0, stage0, ssem0)

                @pl.when(i < half - 1)
                def _():
                    drain_unit(ssem0, stage0)
                    pltpu.sync_copy(idx_hbm.at[pl.ds(u0 + 2 * _U, _U)], eb0)
                    fire_gathers(eb0, stage0, gsem0)

                drain_unit(gsem1, stage1)
                fire_scatters(eb1, stage1, ssem1)
                return carry

            lax.fori_loop(0, half, blk, 0)
            drain_unit(ssem0, stage0)
            drain_unit(ssem1, stage1)
            plsc.subcore_barrier()

            @pl.when(sid < _NS - 1)
            def _copy_full():
                pltpu.sync_copy(acc.at[pl.ds(sid * no, no)],
                                out_hbm.at[cid, f, pl.ds(sid * no, no)])

            @pl.when(sid == _NS - 1)
            def _copy_last():
                pltpu.sync_copy(
                    acc.at[pl.ds((_NS - 1) * no, nlast)],
                    out_hbm.at[cid, f, pl.ds((_NS - 1) * no, nlast)])

            plsc.subcore_barrier()

    run = pl.kernel(
        body,
        out_type=jax.ShapeDtypeStruct((_NC, nf, n, dc), jnp.float32),
        mesh=mesh,
        compiler_params=pltpu.CompilerParams(use_tc_tiling_on_sc=False),
        scratch_types=[
            pltpu.VMEM_SHARED((npad, dc), jnp.float32),
            pltpu.VMEM((_U, 2, _K), jnp.int32),
            pltpu.VMEM((_U, 2, _K), jnp.int32),
            pltpu.VMEM((_U * _K, dc), jnp.float32),
            pltpu.VMEM((_U * _K, dc), jnp.float32),
            pltpu.SemaphoreType.DMA,
            pltpu.SemaphoreType.DMA,
            pltpu.SemaphoreType.DMA,
            pltpu.SemaphoreType.DMA,
        ],
    )
    return run(h3, idx3, zeros_pad)


def _sc_agg(h, idx3, zeros_pad, dc):
    n, d = h.shape
    nf = d // dc
    h3 = h.reshape(n, nf, dc).transpose(1, 0, 2)
    out = _sc_agg_kernel(h3, idx3, zeros_pad, dc)
    return (out[0] + out[1]).transpose(1, 0, 2).reshape(n, d)


def kernel(batch, x, edge_index, pseudo, Wl1, bl1, Wr1, Wl2, bl2, Wr2, Wl3,
           bl3, Wr3, Wl4, bl4, Wr4, Wl5, bl5, Wr5, Wl6, bl6, Wr6, fc1_W,
           fc1_b, fc2_W, fc2_b, bn2_g, bn2_b, fc3_W, fc3_b, out_W, out_b):
    n = x.shape[0]
    e = edge_index.shape[1]
    src = edge_index[0]
    dst = edge_index[1]

    grain = _NC * _NS * 2 * _U * _K
    ep = ((e + grain - 1) // grain) * grain
    padn = ep - e
    pada = jnp.arange(padn, dtype=jnp.int32)
    srcp = jnp.concatenate([src, (pada * 997) % n])
    dstp = jnp.concatenate([dst, n + (pada % _NDUMMY)])
    idx3 = jnp.stack([srcp.reshape(-1, _K), dstp.reshape(-1, _K)], axis=1)
    z16 = jnp.zeros((n + _NPADROWS, 16), jnp.float32)
    z32 = jnp.zeros((n + _NPADROWS, 32), jnp.float32)

    xp = jnp.concatenate(
        [x, jnp.ones((n, 1), jnp.float32), jnp.zeros((n, 2), jnp.float32)],
        axis=1)
    Wl1p = jnp.concatenate([Wl1, jnp.zeros((3, Wl1.shape[1]), jnp.float32)], 0)
    Wr1p = jnp.concatenate([Wr1, jnp.zeros((3, Wr1.shape[1]), jnp.float32)], 0)

    agg1 = _sc_agg(xp, idx3, z16, 16)
    h, cnt = _sage1_tc(agg1, xp, Wl1p, bl1.reshape(1, -1), Wr1p)

    for Wl, bl, Wr, res in (
        (Wl2, bl2, Wr2, False),
        (Wl3, bl3, Wr3, True),
        (Wl4, bl4, Wr4, True),
        (Wl5, bl5, Wr5, True),
        (Wl6, bl6, Wr6, True),
    ):
        agg = _sc_agg(h, idx3, z32, 32)
        h = _sage_tc(agg, h, cnt, Wl, bl.reshape(1, -1), Wr, res)

    h = _mlp_tc(h, fc1_W, fc1_b.reshape(1, -1), True)
    h = _mlp_tc(h, fc2_W, fc2_b.reshape(1, -1), False)

    ps, pc = _pool_tc(h, batch.reshape(-1, 1))
    return _tail_tc(ps, pc, bn2_g.reshape(1, -1), bn2_b.reshape(1, -1),
                    fc3_W, fc3_b.reshape(1, -1), out_W, out_b.reshape(1, -1))


# speedup vs baseline: 6.1586x; 1.1877x over previous
import functools

import jax
import jax.numpy as jnp
import numpy as np
from jax import lax
from jax.experimental import pallas as pl
from jax.experimental.pallas import tpu as pltpu
from jax.experimental.pallas import tpu_sc as plsc

_ROWS = 1000

_NC = 2
_NS = 16
_K = 128
_U = 2
_NPADROWS = 48
_NDUMMY = 16


def _elu(x):
    return jnp.where(x > 0, x, jnp.exp(jnp.minimum(x, 0.0)) - 1.0)


def _sage_body(residual, agg_ref, h_ref, cnt_ref, wl_ref, bl_ref, wr_ref, o_ref):
    inv = 1.0 / jnp.maximum(cnt_ref[...], 1.0)
    mean = (agg_ref[0] + agg_ref[1]) * inv
    t = (jnp.dot(mean, wl_ref[...], preferred_element_type=jnp.float32)
         + bl_ref[...]
         + jnp.dot(h_ref[...], wr_ref[...], preferred_element_type=jnp.float32))
    a = _elu(t)
    o_ref[...] = a + h_ref[...] if residual else a


def _sage_tc(agg, h, cnt, Wl, bl, Wr, residual):
    n, din = h.shape
    dout = Wl.shape[1]
    grid = n // _ROWS
    return pl.pallas_call(
        functools.partial(_sage_body, residual),
        grid=(grid,),
        in_specs=[
            pl.BlockSpec((2, _ROWS, din), lambda i: (0, i, 0)),
            pl.BlockSpec((_ROWS, din), lambda i: (i, 0)),
            pl.BlockSpec((_ROWS, 1), lambda i: (i, 0)),
            pl.BlockSpec((din, dout), lambda i: (0, 0)),
            pl.BlockSpec((1, dout), lambda i: (0, 0)),
            pl.BlockSpec((din, dout), lambda i: (0, 0)),
        ],
        out_specs=pl.BlockSpec((_ROWS, dout), lambda i: (i, 0)),
        out_shape=jax.ShapeDtypeStruct((n, dout), jnp.float32),
    )(agg, h, cnt, Wl, bl, Wr)


def _sage1_body(agg_ref, x_ref, wl_ref, bl_ref, wr_ref, h_ref, cnt_ref):
    agg = agg_ref[0] + agg_ref[1]
    cnt = agg[:, 13:14]
    inv = 1.0 / jnp.maximum(cnt, 1.0)
    mean = agg * inv
    t = (jnp.dot(mean, wl_ref[...], preferred_element_type=jnp.float32)
         + bl_ref[...]
         + jnp.dot(x_ref[...], wr_ref[...], preferred_element_type=jnp.float32))
    h_ref[...] = _elu(t)
    cnt_ref[...] = cnt


def _sage1_tc(agg, xp, Wlp, bl, Wrp):
    n = xp.shape[0]
    dout = Wlp.shape[1]
    grid = n // _ROWS
    return pl.pallas_call(
        _sage1_body,
        grid=(grid,),
        in_specs=[
            pl.BlockSpec((2, _ROWS, 16), lambda i: (0, i, 0)),
            pl.BlockSpec((_ROWS, 16), lambda i: (i, 0)),
            pl.BlockSpec((16, dout), lambda i: (0, 0)),
            pl.BlockSpec((1, dout), lambda i: (0, 0)),
            pl.BlockSpec((16, dout), lambda i: (0, 0)),
        ],
        out_specs=[
            pl.BlockSpec((_ROWS, dout), lambda i: (i, 0)),
            pl.BlockSpec((_ROWS, 1), lambda i: (i, 0)),
        ],
        out_shape=[
            jax.ShapeDtypeStruct((n, dout), jnp.float32),
            jax.ShapeDtypeStruct((n, 1), jnp.float32),
        ],
    )(agg, xp, Wlp, bl, Wrp)


def _mlp_body(residual, h_ref, w_ref, b_ref, o_ref):
    t = _elu(jnp.dot(h_ref[...], w_ref[...], preferred_element_type=jnp.float32)
             + b_ref[...])
    o_ref[...] = t + h_ref[...] if residual else t


def _mlp_tc(h, W, b, residual):
    n, din = h.shape
    dout = W.shape[1]
    grid = n // _ROWS
    return pl.pallas_call(
        functools.partial(_mlp_body, residual),
        grid=(grid,),
        in_specs=[
            pl.BlockSpec((_ROWS, din), lambda i: (i, 0)),
            pl.BlockSpec((din, dout), lambda i: (0, 0)),
            pl.BlockSpec((1, dout), lambda i: (0, 0)),
        ],
        out_specs=pl.BlockSpec((_ROWS, dout), lambda i: (i, 0)),
        out_shape=jax.ShapeDtypeStruct((n, dout), jnp.float32),
    )(h, W, b)


def _pool_body(h_ref, b_ref, ps_ref, pc_ref):
    @pl.when(pl.program_id(0) == 0)
    def _init():
        ps_ref[...] = jnp.zeros_like(ps_ref)
        pc_ref[...] = jnp.zeros_like(pc_ref)

    onehot = (b_ref[...] == lax.broadcasted_iota(jnp.int32, (1, 64), 1)
              ).astype(jnp.float32)
    dn = (((0,), (0,)), ((), ()))
    ps_ref[...] += lax.dot_general(onehot, h_ref[...], dn,
                                   preferred_element_type=jnp.float32)
    pc_ref[...] += lax.dot_general(onehot, jnp.ones_like(h_ref[...]), dn,
                                   preferred_element_type=jnp.float32)


def _pool_tc(h, batch2d):
    n, d = h.shape
    grid = n // _ROWS
    return pl.pallas_call(
        _pool_body,
        grid=(grid,),
        in_specs=[
            pl.BlockSpec((_ROWS, d), lambda i: (i, 0)),
            pl.BlockSpec((_ROWS, 1), lambda i: (i, 0)),
        ],
        out_specs=[
            pl.BlockSpec((64, d), lambda i: (0, 0)),
            pl.BlockSpec((64, d), lambda i: (0, 0)),
        ],
        out_shape=[
            jax.ShapeDtypeStruct((64, d), jnp.float32),
            jax.ShapeDtypeStruct((64, d), jnp.float32),
        ],
    )(h, batch2d)


def _tail_body(ps_ref, pc_ref, g_ref, b_ref, w3_ref, b3_ref, wo_ref, bo_ref,
               y_ref):
    pooled = ps_ref[...] / jnp.maximum(pc_ref[...], 1.0)
    y = pooled * np.float32(1.0 / np.sqrt(1.0 + 1e-5)) * g_ref[...] + b_ref[...]
    y = _elu(jnp.dot(y, w3_ref[...], preferred_element_type=jnp.float32)
             + b3_ref[...])
    y = jnp.dot(y, wo_ref[...], preferred_element_type=jnp.float32) + bo_ref[...]
    nrm = jnp.sqrt(jnp.sum(y * y, axis=-1, keepdims=True))
    y_ref[...] = y / jnp.maximum(nrm, 1e-12)


def _tail_tc(ps, pc, bn2_g, bn2_b, fc3_W, fc3_b, out_W, out_b):
    return pl.pallas_call(
        _tail_body,
        out_shape=jax.ShapeDtypeStruct((64, 3), jnp.float32),
    )(ps, pc, bn2_g, bn2_b, fc3_W, fc3_b, out_W, out_b)


def _sc_agg_kernel(h3, idx3, zeros_pad, dc):
    nf, n, _ = h3.shape
    d = nf * dc
    npad = n + _NPADROWS
    nbt = idx3.shape[0]
    nblk = nbt // (_NC * _NS)
    half = nblk // (2 * _U)
    nz = npad // _NS
    no = npad // _NS
    nlast = n - (_NS - 1) * no

    mesh = plsc.VectorSubcoreMesh(core_axis_name="c", subcore_axis_name="s",
                                  num_cores=_NC, num_subcores=_NS)

    def body(h_hbm, idx_hbm, z_hbm, out_hbm, acc, eb0, eb1, stage0, stage1,
             gsem0, gsem1, ssem0, ssem1):
        cid = lax.axis_index("c")
        sid = lax.axis_index("s")
        bbase = (cid * _NS + sid) * nblk
        for f in range(nf):
            tbl = h_hbm.at[f]
            pltpu.sync_copy(z_hbm.at[pl.ds(sid * nz, nz)],
                            acc.at[pl.ds(sid * nz, nz)])
            plsc.subcore_barrier()

            def drain_unit(sem, stage):
                pltpu.make_async_copy(tbl.at[pl.ds(0, _U * _K)], stage,
                                      sem).wait()

            def fire_gathers(eb, stage, sem):
                for j in range(_U):
                    pltpu.async_copy(tbl.at[eb.at[j, 0]],
                                     stage.at[pl.ds(j * _K, _K)], sem)

            def fire_scatters(eb, stage, sem):
                for j in range(_U):
                    pltpu.async_copy(stage.at[pl.ds(j * _K, _K)],
                                     acc.at[eb.at[j, 1]], sem, add=True)

            pltpu.sync_copy(idx_hbm.at[pl.ds(bbase, _U)], eb0)
            fire_gathers(eb0, stage0, gsem0)

            def blk(i, carry):
                u0 = bbase + 2 * i * _U

                @pl.when(i > 0)
                def _():
                    drain_unit(ssem1, stage1)

                pltpu.sync_copy(idx_hbm.at[pl.ds(u0 + _U, _U)], eb1)
                fire_gathers(eb1, stage1, gsem1)
                drain_unit(gsem0, stage0)
                fire_scatters(eb0, stage0, ssem0)

                @pl.when(i < half - 1)
                def _():
                    drain_unit(ssem0, stage0)
                    pltpu.sync_copy(idx_hbm.at[pl.ds(u0 + 2 * _U, _U)], eb0)
                    fire_gathers(eb0, stage0, gsem0)

                drain_unit(gsem1, stage1)
                fire_scatters(eb1, stage1, ssem1)
                return carry

            lax.fori_loop(0, half, blk, 0)
            drain_unit(ssem0, stage0)
            drain_unit(ssem1, stage1)
            plsc.subcore_barrier()

            if nf > 1:
                def oref(r0, nr):
                    return out_hbm.at[cid, pl.ds(r0, nr), pl.ds(f * dc, dc)]
            else:
                def oref(r0, nr):
                    return out_hbm.at[cid, pl.ds(r0, nr)]

            @pl.when(sid < _NS - 1)
            def _copy_full():
                pltpu.sync_copy(acc.at[pl.ds(sid * no, no)], oref(sid * no, no))

            @pl.when(sid == _NS - 1)
            def _copy_last():
                pltpu.sync_copy(acc.at[pl.ds((_NS - 1) * no, nlast)],
                                oref((_NS - 1) * no, nlast))

            plsc.subcore_barrier()

    run = pl.kernel(
        body,
        out_type=jax.ShapeDtypeStruct((_NC, n, d), jnp.float32),
        mesh=mesh,
        compiler_params=pltpu.CompilerParams(use_tc_tiling_on_sc=False),
        scratch_types=[
            pltpu.VMEM_SHARED((npad, dc), jnp.float32),
            pltpu.VMEM((_U, 2, _K), jnp.int32),
            pltpu.VMEM((_U, 2, _K), jnp.int32),
            pltpu.VMEM((_U * _K, dc), jnp.float32),
            pltpu.VMEM((_U * _K, dc), jnp.float32),
            pltpu.SemaphoreType.DMA,
            pltpu.SemaphoreType.DMA,
            pltpu.SemaphoreType.DMA,
            pltpu.SemaphoreType.DMA,
        ],
    )
    return run(h3, idx3, zeros_pad)


def _sc_agg(h, idx3, zeros_pad, dc):
    n, d = h.shape
    nf = d // dc
    h3 = h.reshape(n, nf, dc).transpose(1, 0, 2) if nf > 1 else h[None]
    return _sc_agg_kernel(h3, idx3, zeros_pad, dc)


def kernel(batch, x, edge_index, pseudo, Wl1, bl1, Wr1, Wl2, bl2, Wr2, Wl3,
           bl3, Wr3, Wl4, bl4, Wr4, Wl5, bl5, Wr5, Wl6, bl6, Wr6, fc1_W,
           fc1_b, fc2_W, fc2_b, bn2_g, bn2_b, fc3_W, fc3_b, out_W, out_b):
    n = x.shape[0]
    e = edge_index.shape[1]
    src = edge_index[0]
    dst = edge_index[1]

    grain = _NC * _NS * 2 * _U * _K
    ep = ((e + grain - 1) // grain) * grain
    padn = ep - e
    pada = jnp.arange(padn, dtype=jnp.int32)
    srcp = jnp.concatenate([src, (pada * 997) % n])
    dstp = jnp.concatenate([dst, n + (pada % _NDUMMY)])
    idx3 = jnp.stack([srcp.reshape(-1, _K), dstp.reshape(-1, _K)], axis=1)
    z16 = jnp.zeros((n + _NPADROWS, 16), jnp.float32)
    z32 = jnp.zeros((n + _NPADROWS, 32), jnp.float32)

    xp = jnp.concatenate(
        [x, jnp.ones((n, 1), jnp.float32), jnp.zeros((n, 2), jnp.float32)],
        axis=1)
    Wl1p = jnp.concatenate([Wl1, jnp.zeros((3, Wl1.shape[1]), jnp.float32)], 0)
    Wr1p = jnp.concatenate([Wr1, jnp.zeros((3, Wr1.shape[1]), jnp.float32)], 0)

    agg1 = _sc_agg(xp, idx3, z16, 16)
    h, cnt = _sage1_tc(agg1, xp, Wl1p, bl1.reshape(1, -1), Wr1p)

    for Wl, bl, Wr, res in (
        (Wl2, bl2, Wr2, False),
        (Wl3, bl3, Wr3, True),
        (Wl4, bl4, Wr4, True),
        (Wl5, bl5, Wr5, True),
        (Wl6, bl6, Wr6, True),
    ):
        agg = _sc_agg(h, idx3, z32, 32)
        h = _sage_tc(agg, h, cnt, Wl, bl.reshape(1, -1), Wr, res)

    h = _mlp_tc(h, fc1_W, fc1_b.reshape(1, -1), True)
    h = _mlp_tc(h, fc2_W, fc2_b.reshape(1, -1), False)

    ps, pc = _pool_tc(h, batch.reshape(-1, 1))
    return _tail_tc(ps, pc, bn2_g.reshape(1, -1), bn2_b.reshape(1, -1),
                    fc3_W, fc3_b.reshape(1, -1), out_W, out_b.reshape(1, -1))


# speedup vs baseline: 7.0617x; 1.1467x over previous
import functools

import jax
import jax.numpy as jnp
import numpy as np
from jax import lax
from jax.experimental import pallas as pl
from jax.experimental.pallas import tpu as pltpu
from jax.experimental.pallas import tpu_sc as plsc

_ROWS = 1000

_NC = 2
_NS = 16
_K = 128
_U = 3
_NPADROWS = 48
_NDUMMY = 16


def _elu(x):
    return jnp.where(x > 0, x, jnp.exp(jnp.minimum(x, 0.0)) - 1.0)


def _sage_body(residual, agg_ref, h_ref, cnt_ref, wl_ref, bl_ref, wr_ref, o_ref):
    inv = 1.0 / jnp.maximum(cnt_ref[...], 1.0)
    mean = (agg_ref[0] + agg_ref[1]) * inv
    t = (jnp.dot(mean, wl_ref[...], preferred_element_type=jnp.float32)
         + bl_ref[...]
         + jnp.dot(h_ref[...], wr_ref[...], preferred_element_type=jnp.float32))
    a = _elu(t)
    o_ref[...] = a + h_ref[...] if residual else a


def _sage_tc(agg, h, cnt, Wl, bl, Wr, residual):
    n, din = h.shape
    dout = Wl.shape[1]
    grid = n // _ROWS
    return pl.pallas_call(
        functools.partial(_sage_body, residual),
        grid=(grid,),
        in_specs=[
            pl.BlockSpec((2, _ROWS, din), lambda i: (0, i, 0)),
            pl.BlockSpec((_ROWS, din), lambda i: (i, 0)),
            pl.BlockSpec((_ROWS, 1), lambda i: (i, 0)),
            pl.BlockSpec((din, dout), lambda i: (0, 0)),
            pl.BlockSpec((1, dout), lambda i: (0, 0)),
            pl.BlockSpec((din, dout), lambda i: (0, 0)),
        ],
        out_specs=pl.BlockSpec((_ROWS, dout), lambda i: (i, 0)),
        out_shape=jax.ShapeDtypeStruct((n, dout), jnp.float32),
    )(agg, h, cnt, Wl, bl, Wr)


def _sage1_body(agg_ref, x_ref, wl_ref, bl_ref, wr_ref, h_ref, cnt_ref):
    agg = agg_ref[0] + agg_ref[1]
    cnt = agg[:, 13:14]
    inv = 1.0 / jnp.maximum(cnt, 1.0)
    mean = agg * inv
    t = (jnp.dot(mean, wl_ref[...], preferred_element_type=jnp.float32)
         + bl_ref[...]
         + jnp.dot(x_ref[...], wr_ref[...], preferred_element_type=jnp.float32))
    h_ref[...] = _elu(t)
    cnt_ref[...] = cnt


def _sage1_tc(agg, xp, Wlp, bl, Wrp):
    n = xp.shape[0]
    dout = Wlp.shape[1]
    grid = n // _ROWS
    return pl.pallas_call(
        _sage1_body,
        grid=(grid,),
        in_specs=[
            pl.BlockSpec((2, _ROWS, 16), lambda i: (0, i, 0)),
            pl.BlockSpec((_ROWS, 16), lambda i: (i, 0)),
            pl.BlockSpec((16, dout), lambda i: (0, 0)),
            pl.BlockSpec((1, dout), lambda i: (0, 0)),
            pl.BlockSpec((16, dout), lambda i: (0, 0)),
        ],
        out_specs=[
            pl.BlockSpec((_ROWS, dout), lambda i: (i, 0)),
            pl.BlockSpec((_ROWS, 1), lambda i: (i, 0)),
        ],
        out_shape=[
            jax.ShapeDtypeStruct((n, dout), jnp.float32),
            jax.ShapeDtypeStruct((n, 1), jnp.float32),
        ],
    )(agg, xp, Wlp, bl, Wrp)


def _mlp_body(residual, h_ref, w_ref, b_ref, o_ref):
    t = _elu(jnp.dot(h_ref[...], w_ref[...], preferred_element_type=jnp.float32)
             + b_ref[...])
    o_ref[...] = t + h_ref[...] if residual else t


def _mlp_tc(h, W, b, residual):
    n, din = h.shape
    dout = W.shape[1]
    grid = n // _ROWS
    return pl.pallas_call(
        functools.partial(_mlp_body, residual),
        grid=(grid,),
        in_specs=[
            pl.BlockSpec((_ROWS, din), lambda i: (i, 0)),
            pl.BlockSpec((din, dout), lambda i: (0, 0)),
            pl.BlockSpec((1, dout), lambda i: (0, 0)),
        ],
        out_specs=pl.BlockSpec((_ROWS, dout), lambda i: (i, 0)),
        out_shape=jax.ShapeDtypeStruct((n, dout), jnp.float32),
    )(h, W, b)


def _pool_body(h_ref, b_ref, ps_ref, pc_ref):
    @pl.when(pl.program_id(0) == 0)
    def _init():
        ps_ref[...] = jnp.zeros_like(ps_ref)
        pc_ref[...] = jnp.zeros_like(pc_ref)

    onehot = (b_ref[...] == lax.broadcasted_iota(jnp.int32, (1, 64), 1)
              ).astype(jnp.float32)
    dn = (((0,), (0,)), ((), ()))
    ps_ref[...] += lax.dot_general(onehot, h_ref[...], dn,
                                   preferred_element_type=jnp.float32)
    pc_ref[...] += lax.dot_general(onehot, jnp.ones_like(h_ref[...]), dn,
                                   preferred_element_type=jnp.float32)


def _pool_tc(h, batch2d):
    n, d = h.shape
    grid = n // _ROWS
    return pl.pallas_call(
        _pool_body,
        grid=(grid,),
        in_specs=[
            pl.BlockSpec((_ROWS, d), lambda i: (i, 0)),
            pl.BlockSpec((_ROWS, 1), lambda i: (i, 0)),
        ],
        out_specs=[
            pl.BlockSpec((64, d), lambda i: (0, 0)),
            pl.BlockSpec((64, d), lambda i: (0, 0)),
        ],
        out_shape=[
            jax.ShapeDtypeStruct((64, d), jnp.float32),
            jax.ShapeDtypeStruct((64, d), jnp.float32),
        ],
    )(h, batch2d)


def _tail_body(ps_ref, pc_ref, g_ref, b_ref, w3_ref, b3_ref, wo_ref, bo_ref,
               y_ref):
    pooled = ps_ref[...] / jnp.maximum(pc_ref[...], 1.0)
    y = pooled * np.float32(1.0 / np.sqrt(1.0 + 1e-5)) * g_ref[...] + b_ref[...]
    y = _elu(jnp.dot(y, w3_ref[...], preferred_element_type=jnp.float32)
             + b3_ref[...])
    y = jnp.dot(y, wo_ref[...], preferred_element_type=jnp.float32) + bo_ref[...]
    nrm = jnp.sqrt(jnp.sum(y * y, axis=-1, keepdims=True))
    y_ref[...] = y / jnp.maximum(nrm, 1e-12)


def _tail_tc(ps, pc, bn2_g, bn2_b, fc3_W, fc3_b, out_W, out_b):
    return pl.pallas_call(
        _tail_body,
        out_shape=jax.ShapeDtypeStruct((64, 3), jnp.float32),
    )(ps, pc, bn2_g, bn2_b, fc3_W, fc3_b, out_W, out_b)


def _sc_agg_kernel(h3, idx3, zeros_pad, dc):
    nf, n, _ = h3.shape
    d = nf * dc
    npad = n + _NPADROWS
    nbt = idx3.shape[0]
    nblk = nbt // (_NC * _NS)
    half = nblk // (2 * _U)
    nz = npad // _NS
    no = npad // _NS
    nlast = n - (_NS - 1) * no

    mesh = plsc.VectorSubcoreMesh(core_axis_name="c", subcore_axis_name="s",
                                  num_cores=_NC, num_subcores=_NS)

    def body(h_hbm, idx_hbm, z_hbm, out_hbm, acc, eb0, eb1, stage0, stage1,
             gsem0, gsem1, ssem0, ssem1):
        cid = lax.axis_index("c")
        sid = lax.axis_index("s")
        bbase = (cid * _NS + sid) * nblk
        for f in range(nf):
            tbl = h_hbm.at[f]
            pltpu.sync_copy(z_hbm.at[pl.ds(sid * nz, nz)],
                            acc.at[pl.ds(sid * nz, nz)])
            plsc.subcore_barrier()

            def drain_unit(sem, stage):
                pltpu.make_async_copy(tbl.at[pl.ds(0, _U * _K)], stage,
                                      sem).wait()

            def fire_gathers(eb, stage, sem):
                for j in range(_U):
                    pltpu.async_copy(tbl.at[eb.at[j, 0]],
                                     stage.at[pl.ds(j * _K, _K)], sem)

            def fire_scatters(eb, stage, sem):
                for j in range(_U):
                    pltpu.async_copy(stage.at[pl.ds(j * _K, _K)],
                                     acc.at[eb.at[j, 1]], sem, add=True)

            pltpu.sync_copy(idx_hbm.at[pl.ds(bbase, _U)], eb0)
            fire_gathers(eb0, stage0, gsem0)

            def blk(i, carry):
                u0 = bbase + 2 * i * _U

                @pl.when(i > 0)
                def _():
                    drain_unit(ssem1, stage1)

                pltpu.sync_copy(idx_hbm.at[pl.ds(u0 + _U, _U)], eb1)
                fire_gathers(eb1, stage1, gsem1)
                drain_unit(gsem0, stage0)
                fire_scatters(eb0, stage0, ssem0)

                @pl.when(i < half - 1)
                def _():
                    drain_unit(ssem0, stage0)
                    pltpu.sync_copy(idx_hbm.at[pl.ds(u0 + 2 * _U, _U)], eb0)
                    fire_gathers(eb0, stage0, gsem0)

                drain_unit(gsem1, stage1)
                fire_scatters(eb1, stage1, ssem1)
                return carry

            lax.fori_loop(0, half, blk, 0)
            drain_unit(ssem0, stage0)
            drain_unit(ssem1, stage1)
            plsc.subcore_barrier()

            if nf > 1:
                def oref(r0, nr):
                    return out_hbm.at[cid, pl.ds(r0, nr), pl.ds(f * dc, dc)]
            else:
                def oref(r0, nr):
                    return out_hbm.at[cid, pl.ds(r0, nr)]

            @pl.when(sid < _NS - 1)
            def _copy_full():
                pltpu.sync_copy(acc.at[pl.ds(sid * no, no)], oref(sid * no, no))

            @pl.when(sid == _NS - 1)
            def _copy_last():
                pltpu.sync_copy(acc.at[pl.ds((_NS - 1) * no, nlast)],
                                oref((_NS - 1) * no, nlast))

            plsc.subcore_barrier()

    run = pl.kernel(
        body,
        out_type=jax.ShapeDtypeStruct((_NC, n, d), jnp.float32),
        mesh=mesh,
        compiler_params=pltpu.CompilerParams(use_tc_tiling_on_sc=False),
        scratch_types=[
            pltpu.VMEM_SHARED((npad, dc), jnp.float32),
            pltpu.VMEM((_U, 2, _K), jnp.int32),
            pltpu.VMEM((_U, 2, _K), jnp.int32),
            pltpu.VMEM((_U * _K, dc), jnp.float32),
            pltpu.VMEM((_U * _K, dc), jnp.float32),
            pltpu.SemaphoreType.DMA,
            pltpu.SemaphoreType.DMA,
            pltpu.SemaphoreType.DMA,
            pltpu.SemaphoreType.DMA,
        ],
    )
    return run(h3, idx3, zeros_pad)


def _sc_agg(h, idx3, zeros_pad, dc):
    n, d = h.shape
    nf = d // dc
    h3 = h.reshape(n, nf, dc).transpose(1, 0, 2) if nf > 1 else h[None]
    return _sc_agg_kernel(h3, idx3, zeros_pad, dc)


def kernel(batch, x, edge_index, pseudo, Wl1, bl1, Wr1, Wl2, bl2, Wr2, Wl3,
           bl3, Wr3, Wl4, bl4, Wr4, Wl5, bl5, Wr5, Wl6, bl6, Wr6, fc1_W,
           fc1_b, fc2_W, fc2_b, bn2_g, bn2_b, fc3_W, fc3_b, out_W, out_b):
    n = x.shape[0]
    e = edge_index.shape[1]
    src = edge_index[0]
    dst = edge_index[1]

    grain = _NC * _NS * 2 * _U * _K
    ep = ((e + grain - 1) // grain) * grain
    padn = ep - e
    pada = jnp.arange(padn, dtype=jnp.int32)
    srcp = jnp.concatenate([src, (pada * 997) % n])
    dstp = jnp.concatenate([dst, n + (pada % _NDUMMY)])
    idx3 = jnp.stack([srcp.reshape(-1, _K), dstp.reshape(-1, _K)], axis=1)
    z16 = jnp.zeros((n + _NPADROWS, 16), jnp.float32)
    z32 = jnp.zeros((n + _NPADROWS, 32), jnp.float32)

    xp = jnp.concatenate(
        [x, jnp.ones((n, 1), jnp.float32), jnp.zeros((n, 2), jnp.float32)],
        axis=1)
    Wl1p = jnp.concatenate([Wl1, jnp.zeros((3, Wl1.shape[1]), jnp.float32)], 0)
    Wr1p = jnp.concatenate([Wr1, jnp.zeros((3, Wr1.shape[1]), jnp.float32)], 0)

    agg1 = _sc_agg(xp, idx3, z16, 16)
    h, cnt = _sage1_tc(agg1, xp, Wl1p, bl1.reshape(1, -1), Wr1p)

    for Wl, bl, Wr, res in (
        (Wl2, bl2, Wr2, False),
        (Wl3, bl3, Wr3, True),
        (Wl4, bl4, Wr4, True),
        (Wl5, bl5, Wr5, True),
        (Wl6, bl6, Wr6, True),
    ):
        agg = _sc_agg(h, idx3, z32, 32)
        h = _sage_tc(agg, h, cnt, Wl, bl.reshape(1, -1), Wr, res)

    h = _mlp_tc(h, fc1_W, fc1_b.reshape(1, -1), True)
    h = _mlp_tc(h, fc2_W, fc2_b.reshape(1, -1), False)

    ps, pc = _pool_tc(h, batch.reshape(-1, 1))
    return _tail_tc(ps, pc, bn2_g.reshape(1, -1), bn2_b.reshape(1, -1),
                    fc3_W, fc3_b.reshape(1, -1), out_W, out_b.reshape(1, -1))
